# Initial kernel scaffold; baseline (speedup 1.0000x reference)
import functools

import jax
import jax.numpy as jnp
from jax import lax
from jax.experimental import pallas as pl
from jax.experimental.pallas import tpu as pltpu
from jax.experimental.pallas import tpu_sc as plsc

NC = 2
NS = 16
NW = NC * NS
LANES = 16

WDIM = 64
TDIM = 32
NTAGS = 50

BLK = 128
BLKS_PER_GRP = 8
GRP = BLK * BLKS_PER_GRP


def _body(n_groups, idx_hbm, wtab_hbm, ttab_hbm, out_hbm,
          idx_v, tag_v, wbuf, tbuf, idx_sem, wsem, tsem, osem):
    wid = lax.axis_index("s") * NC + lax.axis_index("c")
    grp_base = wid * n_groups * BLKS_PER_GRP

    def group(g, carry):
        row0 = grp_base + g * BLKS_PER_GRP
        tok0 = row0 * BLK

        pltpu.async_copy(idx_hbm.at[pl.ds(row0, BLKS_PER_GRP)], idx_v,
                         idx_sem).wait()

        for j in range(BLKS_PER_GRP):
            for c in range(BLK // LANES):
                sl = pl.ds(c * LANES, LANES)
                tag_v[j, sl] = lax.rem(idx_v[j, sl], jnp.int32(NTAGS))

        copies = []
        for j in range(BLKS_PER_GRP):
            copies.append(pltpu.async_copy(
                wtab_hbm.at[idx_v.at[j]],
                wbuf.at[pl.ds(j * BLK, BLK)], wsem))
            copies.append(pltpu.async_copy(
                ttab_hbm.at[tag_v.at[j]],
                tbuf.at[pl.ds(j * BLK, BLK)], tsem))
        for cp in copies:
            cp.wait()

        ow = pltpu.async_copy(wbuf, out_hbm.at[pl.ds(tok0, GRP),
                                               pl.ds(0, WDIM)], osem)
        ot = pltpu.async_copy(tbuf, out_hbm.at[pl.ds(tok0, GRP),
                                               pl.ds(WDIM, TDIM)], osem)
        ow.wait()
        ot.wait()
        return carry

    lax.fori_loop(0, n_groups, group, 0)


def kernel(sentence, word_table, tag_table):
    B, L = sentence.shape
    n = B * L
    assert n % (NW * GRP) == 0
    n_groups = n // (NW * GRP)

    idx = sentence.reshape(n // BLK, BLK).astype(jnp.int32)

    mesh = plsc.VectorSubcoreMesh(core_axis_name="c", subcore_axis_name="s")
    out = pl.kernel(
        functools.partial(_body, n_groups),
        out_type=jax.ShapeDtypeStruct((n, WDIM + TDIM), jnp.float32),
        mesh=mesh,
        scratch_types=[
            pltpu.VMEM((BLKS_PER_GRP, BLK), jnp.int32),
            pltpu.VMEM((BLKS_PER_GRP, BLK), jnp.int32),
            pltpu.VMEM((GRP, WDIM), jnp.float32),
            pltpu.VMEM((GRP, TDIM), jnp.float32),
            pltpu.SemaphoreType.DMA,
            pltpu.SemaphoreType.DMA,
            pltpu.SemaphoreType.DMA,
            pltpu.SemaphoreType.DMA,
        ],
    )(idx, word_table, tag_table)
    return out.reshape(B, L, WDIM + TDIM)


# speedup vs baseline: 1.5126x; 1.5126x over previous
import functools

import jax
import jax.numpy as jnp
from jax import lax
from jax.experimental import pallas as pl
from jax.experimental.pallas import tpu as pltpu
from jax.experimental.pallas import tpu_sc as plsc

NC = 2
NS = 16
NW = NC * NS
LANES = 16

WDIM = 64
TDIM = 32
NTAGS = 50

BLK = 128
BLKS_PER_GRP = 8
GRP = BLK * BLKS_PER_GRP


def _body(n_groups, idx_hbm, wtab_hbm, ttab_hbm, out_hbm,
          idx_v, tag_v, wbuf, tbuf, idx_sem, wsem, tsem, osem):
    wid = lax.axis_index("s") * NC + lax.axis_index("c")
    grp_base = wid * n_groups * BLKS_PER_GRP

    def group(g, carry):
        row0 = grp_base + g * BLKS_PER_GRP
        tok0 = row0 * BLK

        pltpu.async_copy(idx_hbm.at[pl.ds(row0, BLKS_PER_GRP)], idx_v,
                         idx_sem).wait()

        for j in range(BLKS_PER_GRP):
            for c in range(BLK // LANES):
                sl = pl.ds(c * LANES, LANES)
                tag_v[j, sl] = lax.rem(idx_v[j, sl], jnp.int32(NTAGS))

        copies = []
        for j in range(BLKS_PER_GRP):
            copies.append(pltpu.async_copy(
                wtab_hbm.at[idx_v.at[j]],
                wbuf.at[pl.ds(j * BLK, BLK)], wsem))
            copies.append(pltpu.async_copy(
                ttab_hbm.at[tag_v.at[j]],
                tbuf.at[pl.ds(j * BLK, BLK)], tsem))
        for cp in copies:
            cp.wait()

        ow = pltpu.async_copy(wbuf, out_hbm.at[pl.ds(tok0, GRP),
                                               pl.ds(0, WDIM)], osem)
        ot = pltpu.async_copy(tbuf, out_hbm.at[pl.ds(tok0, GRP),
                                               pl.ds(WDIM, TDIM)], osem)
        ow.wait()
        ot.wait()
        return carry

    lax.fori_loop(0, n_groups, group, 0)


def kernel(sentence, word_table, tag_table):
    B, L = sentence.shape
    n = B * L
    assert n % (NW * GRP) == 0
    n_groups = n // (NW * GRP)

    idx = sentence.reshape(n // BLK, BLK).astype(jnp.int32)

    mesh = plsc.VectorSubcoreMesh(core_axis_name="c", subcore_axis_name="s")
    out = pl.kernel(
        functools.partial(_body, n_groups),
        out_type=jax.ShapeDtypeStruct((n, WDIM + TDIM), jnp.float32),
        mesh=mesh,
        compiler_params=pltpu.CompilerParams(use_tc_tiling_on_sc=False),
        scratch_types=[
            pltpu.VMEM((BLKS_PER_GRP, BLK), jnp.int32),
            pltpu.VMEM((BLKS_PER_GRP, BLK), jnp.int32),
            pltpu.VMEM((GRP, WDIM), jnp.float32),
            pltpu.VMEM((GRP, TDIM), jnp.float32),
            pltpu.SemaphoreType.DMA,
            pltpu.SemaphoreType.DMA,
            pltpu.SemaphoreType.DMA,
            pltpu.SemaphoreType.DMA,
        ],
    )(idx, word_table, tag_table)
    return out.reshape(B, L, WDIM + TDIM)


# speedup vs baseline: 1.5185x; 1.0039x over previous
import functools

import jax
import jax.numpy as jnp
from jax import lax
from jax.experimental import pallas as pl
from jax.experimental.pallas import tpu as pltpu
from jax.experimental.pallas import tpu_sc as plsc

NC = 2
NS = 16
NW = NC * NS
LANES = 16

WDIM = 64
TDIM = 32
NTAGS = 50

BLK = 128
BPG = 4
GRP = BLK * BPG


def _tag_ids(iv):
    f = iv.astype(jnp.float32) * jnp.float32(1.0 / NTAGS)
    q = f.astype(jnp.int32)
    r = iv - q * jnp.int32(NTAGS)
    return jnp.where(r >= NTAGS, r - jnp.int32(NTAGS), r)


def _body(T, idx_hbm, wtab_hbm, ttab_hbm, out_hbm,
          idx_v, tag_v, wbuf, tbuf,
          isem0, isem1, gsem0, gsem1, osem0, osem1):
    isem = (isem0, isem1)
    gsem = (gsem0, gsem1)
    osem = (osem0, osem1)
    wid = lax.axis_index("s") * NC + lax.axis_index("c")
    base_row = wid * (2 * T * BPG)

    def out_slices(tok0):
        return (out_hbm.at[pl.ds(tok0, GRP), pl.ds(0, WDIM)],
                out_hbm.at[pl.ds(tok0, GRP), pl.ds(WDIM, TDIM)])

    for p in (0, 1):
        pltpu.async_copy(idx_hbm.at[pl.ds(base_row + p * BPG, BPG)],
                         idx_v.at[p], isem[p])

    def dbl(t, carry):
        for p in (0, 1):
            row0 = base_row + (2 * t + p) * BPG
            tok0 = row0 * BLK

            pltpu.make_async_copy(idx_hbm.at[pl.ds(row0, BPG)],
                                  idx_v.at[p], isem[p]).wait()

            for j in range(BPG):
                for c in range(BLK // LANES):
                    sl = pl.ds(c * LANES, LANES)
                    tag_v[p, j, sl] = _tag_ids(idx_v[p, j, sl])

            ows, ots = out_slices(tok0)

            @pl.when(t > 0)
            def _():
                pltpu.make_async_copy(wbuf.at[p], ows, osem[p]).wait()
                pltpu.make_async_copy(tbuf.at[p], ots, osem[p]).wait()

            copies = []
            for j in range(BPG):
                copies.append(pltpu.async_copy(
                    wtab_hbm.at[idx_v.at[p, j]],
                    wbuf.at[p, pl.ds(j * BLK, BLK)], gsem[p]))
                copies.append(pltpu.async_copy(
                    ttab_hbm.at[tag_v.at[p, j]],
                    tbuf.at[p, pl.ds(j * BLK, BLK)], gsem[p]))

            for cp in copies:
                cp.wait()

            @pl.when(t < T - 1)
            def _():
                pltpu.async_copy(
                    idx_hbm.at[pl.ds(row0 + 2 * BPG, BPG)],
                    idx_v.at[p], isem[p])

            pltpu.async_copy(wbuf.at[p], ows, osem[p])
            pltpu.async_copy(tbuf.at[p], ots, osem[p])
        return carry

    lax.fori_loop(0, T, dbl, 0)

    for p in (0, 1):
        row0 = base_row + (2 * (T - 1) + p) * BPG
        ows, ots = out_slices(row0 * BLK)
        pltpu.make_async_copy(wbuf.at[p], ows, osem[p]).wait()
        pltpu.make_async_copy(tbuf.at[p], ots, osem[p]).wait()


def kernel(sentence, word_table, tag_table):
    B, L = sentence.shape
    n = B * L
    assert n % (NW * 2 * GRP) == 0
    T = n // (NW * 2 * GRP)

    idx = sentence.reshape(n // BLK, BLK).astype(jnp.int32)

    mesh = plsc.VectorSubcoreMesh(core_axis_name="c", subcore_axis_name="s")
    out = pl.kernel(
        functools.partial(_body, T),
        out_type=jax.ShapeDtypeStruct((n, WDIM + TDIM), jnp.float32),
        mesh=mesh,
        compiler_params=pltpu.CompilerParams(use_tc_tiling_on_sc=False),
        scratch_types=[
            pltpu.VMEM((2, BPG, BLK), jnp.int32),
            pltpu.VMEM((2, BPG, BLK), jnp.int32),
            pltpu.VMEM((2, GRP, WDIM), jnp.float32),
            pltpu.VMEM((2, GRP, TDIM), jnp.float32),
        ] + [pltpu.SemaphoreType.DMA] * 6,
    )(idx, word_table, tag_table)
    return out.reshape(B, L, WDIM + TDIM)
